# baseline (speedup 1.0000x reference)
import functools

import jax
import jax.numpy as jnp
from jax import lax
from jax.experimental import pallas as pl
from jax.experimental.pallas import tpu as pltpu
from jax.experimental.pallas import tpu_sc as plsc

N = 10000
E = 160000
NCORE = 2
NSUB = 16
K = 50
NBUF = 4
PH = 40
NPASS = E // NSUB // K // PH
PH2 = 20
NPASS2 = E // (NCORE * NSUB) // K // PH2
R_MAIN = 624
R_LAST = N - (NSUB - 1) * R_MAIN
BN = 1000


def _mm_call(h, wcat, b2, split):
    Din = h.shape[1]
    Dout = wcat.shape[1] // 2
    S = Dout // 2

    def body(h_ref, w_ref, b_ref, *outs):
        out = jnp.dot(h_ref[...], w_ref[...],
                      preferred_element_type=jnp.float32)
        p = out[:, :Dout]
        q = out[:, Dout:] + b_ref[...]
        if split:
            outs[0][...] = p[:, :S]
            outs[1][...] = p[:, S:]
            outs[2][...] = q[:, :S]
            outs[3][...] = q[:, S:]
        else:
            outs[0][...] = p
            outs[1][...] = q

    n_out, w_out = (4, S) if split else (2, Dout)
    os_ = jax.ShapeDtypeStruct((N, w_out), jnp.float32)
    return pl.pallas_call(
        body,
        grid=(N // BN,),
        in_specs=[
            pl.BlockSpec((BN, Din), lambda i: (i, 0)),
            pl.BlockSpec((Din, 2 * Dout), lambda i: (0, 0)),
            pl.BlockSpec((1, Dout), lambda i: (0, 0)),
        ],
        out_specs=[pl.BlockSpec((BN, w_out), lambda i: (i, 0))] * n_out,
        out_shape=[os_] * n_out,
    )(h, wcat, b2)


def _edge_loop_ring(tbl, srcv, dstv, acc, rowbufs, sems, n_chunks):
    for b in range(NBUF):
        pltpu.async_copy(tbl.at[srcv.at[b]], rowbufs[b], sems[b])

    def body(t, carry):
        base = t * NBUF
        for b in range(NBUF):
            j = base + b
            pltpu.make_async_copy(tbl.at[srcv.at[0]],
                                  rowbufs[b], sems[b]).wait()
            pltpu.sync_copy(rowbufs[b], acc.at[dstv.at[j]], add=True)

            @pl.when(j + NBUF < n_chunks)
            def _():
                pltpu.async_copy(tbl.at[srcv.at[j + NBUF]],
                                 rowbufs[b], sems[b])
        return carry

    lax.fori_loop(0, n_chunks // NBUF, body, 0)


@functools.cache
def _make_sc(S):
    mesh = plsc.VectorSubcoreMesh(
        core_axis_name="c", subcore_axis_name="s",
        num_cores=NCORE, num_subcores=NSUB)

    @functools.partial(
        pl.kernel,
        out_type=[jax.ShapeDtypeStruct((N, S), jnp.float32)] * 2,
        mesh=mesh,
        scratch_types=[
            pltpu.VMEM((PH, K), jnp.int32),
            pltpu.VMEM((PH, K), jnp.int32),
        ] + [pltpu.VMEM((K, S), jnp.float32)] * NBUF + [
            pltpu.VMEM_SHARED((N, S), jnp.float32),
        ] + [pltpu.SemaphoreType.DMA] * NBUF,
    )
    def sc(p0, p1, q0, q1, srcr, dstr, out0, out1,
           srcv, dstv, *rest):
        rowbufs = rest[:NBUF]
        acc = rest[NBUF]
        sems = rest[NBUF + 1:]
        c = lax.axis_index("c")
        s = lax.axis_index("s")
        r0 = pl.multiple_of(s * R_MAIN, 8)

        def rowwise(fn):
            @pl.when(s < NSUB - 1)
            def _():
                fn(r0, R_MAIN)

            @pl.when(s == NSUB - 1)
            def _():
                fn((NSUB - 1) * R_MAIN, R_LAST)

        def init(row0, nrows):
            @pl.when(c == 0)
            def _():
                pltpu.sync_copy(q0.at[pl.ds(row0, nrows)],
                                acc.at[pl.ds(row0, nrows)])

            @pl.when(c == 1)
            def _():
                pltpu.sync_copy(q1.at[pl.ds(row0, nrows)],
                                acc.at[pl.ds(row0, nrows)])

        rowwise(init)
        plsc.subcore_barrier()

        for ps in range(NPASS):
            pltpu.sync_copy(srcr.at[s, ps], srcv)
            pltpu.sync_copy(dstr.at[s, ps], dstv)

            @pl.when(c == 0)
            def _():
                _edge_loop_ring(p0, srcv, dstv, acc, rowbufs, sems, PH)

            @pl.when(c == 1)
            def _():
                _edge_loop_ring(p1, srcv, dstv, acc, rowbufs, sems, PH)

        plsc.subcore_barrier()

        def writeout(row0, nrows):
            @pl.when(c == 0)
            def _():
                pltpu.sync_copy(acc.at[pl.ds(row0, nrows)],
                                out0.at[pl.ds(row0, nrows)])

            @pl.when(c == 1)
            def _():
                pltpu.sync_copy(acc.at[pl.ds(row0, nrows)],
                                out1.at[pl.ds(row0, nrows)])

        rowwise(writeout)

    return sc


@functools.cache
def _make_sc_last():
    D = 128
    mesh = plsc.VectorSubcoreMesh(
        core_axis_name="c", subcore_axis_name="s",
        num_cores=NCORE, num_subcores=NSUB)

    @functools.partial(
        pl.kernel,
        out_type=[jax.ShapeDtypeStruct((N, D), jnp.float32)] * 2,
        mesh=mesh,
        scratch_types=[
            pltpu.VMEM((PH2, K), jnp.int32),
            pltpu.VMEM((PH2, K), jnp.int32),
        ] + [pltpu.VMEM((K, D), jnp.float32)] * NBUF + [
            pltpu.VMEM_SHARED((N, D), jnp.float32),
        ] + [pltpu.SemaphoreType.DMA] * NBUF,
    )
    def sc(p, q, z, srcr, dstr, out0, out1, srcv, dstv, *rest):
        rowbufs = rest[:NBUF]
        acc = rest[NBUF]
        sems = rest[NBUF + 1:]
        c = lax.axis_index("c")
        s = lax.axis_index("s")
        r0 = pl.multiple_of(s * R_MAIN, 8)
        w = c * NSUB + s

        def rowwise(fn):
            @pl.when(s < NSUB - 1)
            def _():
                fn(r0, R_MAIN)

            @pl.when(s == NSUB - 1)
            def _():
                fn((NSUB - 1) * R_MAIN, R_LAST)


        def init(row0, nrows):
            @pl.when(c == 0)
            def _():
                pltpu.sync_copy(q.at[pl.ds(row0, nrows)],
                                acc.at[pl.ds(row0, nrows)])

            @pl.when(c == 1)
            def _():
                pltpu.sync_copy(z.at[pl.ds(row0, nrows)],
                                acc.at[pl.ds(row0, nrows)])

        rowwise(init)
        plsc.subcore_barrier()

        for ps in range(NPASS2):
            pltpu.sync_copy(srcr.at[w, ps], srcv)
            pltpu.sync_copy(dstr.at[w, ps], dstv)
            _edge_loop_ring(p, srcv, dstv, acc, rowbufs, sems, PH2)

        plsc.subcore_barrier()

        def writeout(row0, nrows):
            @pl.when(c == 0)
            def _():
                pltpu.sync_copy(acc.at[pl.ds(row0, nrows)],
                                out0.at[pl.ds(row0, nrows)])

            @pl.when(c == 1)
            def _():
                pltpu.sync_copy(acc.at[pl.ds(row0, nrows)],
                                out1.at[pl.ds(row0, nrows)])

        rowwise(writeout)

    return sc


def _fin_call(o0, o1, last):
    if last:
        Dout = o0.shape[1]
    else:
        Dout = o0.shape[1] * 2

    def body(a_ref, b_ref, o_ref):
        if last:
            v = a_ref[...] + b_ref[...]
        else:
            v = jnp.concatenate([a_ref[...], b_ref[...]], axis=1)
        nrm = jnp.sqrt(jnp.sum(v * v, axis=1, keepdims=True))
        v = v / jnp.maximum(nrm, 1e-12)
        if last:
            m = jnp.max(v, axis=1, keepdims=True)
            e = v - m
            o_ref[...] = e - jnp.log(jnp.sum(jnp.exp(e), axis=1,
                                             keepdims=True))
        else:
            o_ref[...] = jnp.maximum(v, 0.0)

    Sin = o0.shape[1]
    return pl.pallas_call(
        body,
        grid=(N // BN,),
        in_specs=[pl.BlockSpec((BN, Sin), lambda i: (i, 0))] * 2,
        out_specs=pl.BlockSpec((BN, Dout), lambda i: (i, 0)),
        out_shape=jax.ShapeDtypeStruct((N, Dout), jnp.float32),
    )(o0, o1)


def kernel(x, edge_index, Wl0, bl0, Wr0, Wl1, bl1, Wr1,
           Wl2, bl2, Wr2, Wl3, bl3, Wr3):
    src = edge_index[0].reshape(NSUB, NPASS, PH, K)
    dst = edge_index[1].reshape(NSUB, NPASS, PH, K)
    src2 = edge_index[0].reshape(NCORE * NSUB, NPASS2, PH2, K)
    dst2 = edge_index[1].reshape(NCORE * NSUB, NPASS2, PH2, K)
    zeros = jnp.zeros((N, 128), jnp.float32)
    layers = [(Wl0, bl0, Wr0), (Wl1, bl1, Wr1), (Wl2, bl2, Wr2),
              (Wl3, bl3, Wr3)]
    h = x
    for i, (Wl, bl, Wr) in enumerate(layers):
        Dout = Wl.shape[1]
        last = i == len(layers) - 1
        wcat = jnp.concatenate([Wl, Wr], axis=1)
        b2 = bl.reshape(1, Dout)
        if last:
            p, q = _mm_call(h, wcat, b2, split=False)
            out0, out1 = _make_sc_last()(p, q, zeros, src2, dst2)
        else:
            p0, p1, q0, q1 = _mm_call(h, wcat, b2, split=True)
            out0, out1 = _make_sc(Dout // 2)(p0, p1, q0, q1, src, dst)
        h = _fin_call(out0, out1, last=last)
    return h


# speedup vs baseline: 1.0782x; 1.0782x over previous
import functools

import jax
import jax.numpy as jnp
from jax import lax
from jax.experimental import pallas as pl
from jax.experimental.pallas import tpu as pltpu
from jax.experimental.pallas import tpu_sc as plsc

N = 10000
E = 160000
NCORE = 2
NSUB = 16
K = 125
PH = 40
NPASS = E // NSUB // K // PH
CH2 = E // (NCORE * NSUB) // K
R_MAIN = 624
R_LAST = N - (NSUB - 1) * R_MAIN
BN = 1000


def _mm_call(h, wcat, b2, split):
    Din = h.shape[1]
    Dout = wcat.shape[1] // 2
    S = Dout // 2

    def body(h_ref, w_ref, b_ref, *outs):
        out = jnp.dot(h_ref[...], w_ref[...],
                      preferred_element_type=jnp.float32)
        p = out[:, :Dout]
        q = out[:, Dout:] + b_ref[...]
        if split:
            outs[0][...] = p[:, :S]
            outs[1][...] = p[:, S:]
            outs[2][...] = q[:, :S]
            outs[3][...] = q[:, S:]
        else:
            outs[0][...] = p
            outs[1][...] = q

    n_out, w_out = (4, S) if split else (2, Dout)
    os_ = jax.ShapeDtypeStruct((N, w_out), jnp.float32)
    return pl.pallas_call(
        body,
        grid=(N // BN,),
        in_specs=[
            pl.BlockSpec((BN, Din), lambda i: (i, 0)),
            pl.BlockSpec((Din, 2 * Dout), lambda i: (0, 0)),
            pl.BlockSpec((1, Dout), lambda i: (0, 0)),
        ],
        out_specs=[pl.BlockSpec((BN, w_out), lambda i: (i, 0))] * n_out,
        out_shape=[os_] * n_out,
    )(h, wcat, b2)


def _fused_call(o0, o1, wcat, b2, split):
    Dh = o0.shape[1] * 2
    Dout = wcat.shape[1] // 2
    S = Dout // 2

    def body(a_ref, b_ref, w_ref, bias_ref, *outs):
        v = jnp.concatenate([a_ref[...], b_ref[...]], axis=1)
        nrm = jnp.sqrt(jnp.sum(v * v, axis=1, keepdims=True))
        v = jnp.maximum(v / jnp.maximum(nrm, 1e-12), 0.0)
        out = jnp.dot(v, w_ref[...], preferred_element_type=jnp.float32)
        p = out[:, :Dout]
        q = out[:, Dout:] + bias_ref[...]
        if split:
            outs[0][...] = p[:, :S]
            outs[1][...] = p[:, S:]
            outs[2][...] = q[:, :S]
            outs[3][...] = q[:, S:]
        else:
            outs[0][...] = p
            outs[1][...] = q

    n_out, w_out = (4, S) if split else (2, Dout)
    os_ = jax.ShapeDtypeStruct((N, w_out), jnp.float32)
    return pl.pallas_call(
        body,
        grid=(N // BN,),
        in_specs=[
            pl.BlockSpec((BN, Dh // 2), lambda i: (i, 0)),
            pl.BlockSpec((BN, Dh // 2), lambda i: (i, 0)),
            pl.BlockSpec((Dh, 2 * Dout), lambda i: (0, 0)),
            pl.BlockSpec((1, Dout), lambda i: (0, 0)),
        ],
        out_specs=[pl.BlockSpec((BN, w_out), lambda i: (i, 0))] * n_out,
        out_shape=[os_] * n_out,
    )(o0, o1, wcat, b2)


def _edge_loop_db(tbl, srcv, dstv, acc, rows0, rows1, sem0, sem1, n_chunks):
    pltpu.async_copy(tbl.at[srcv.at[0]], rows0, sem0)

    def body(t, carry):
        a = 2 * t
        pltpu.async_copy(tbl.at[srcv.at[a + 1]], rows1, sem1)
        pltpu.make_async_copy(tbl.at[srcv.at[0]], rows0, sem0).wait()
        pltpu.sync_copy(rows0, acc.at[dstv.at[a]], add=True)

        @pl.when(t < n_chunks // 2 - 1)
        def _():
            pltpu.async_copy(tbl.at[srcv.at[a + 2]], rows0, sem0)

        pltpu.make_async_copy(tbl.at[srcv.at[0]], rows1, sem1).wait()
        pltpu.sync_copy(rows1, acc.at[dstv.at[a + 1]], add=True)
        return carry

    lax.fori_loop(0, n_chunks // 2, body, 0)


@functools.cache
def _make_sc(S):
    mesh = plsc.VectorSubcoreMesh(
        core_axis_name="c", subcore_axis_name="s",
        num_cores=NCORE, num_subcores=NSUB)

    @functools.partial(
        pl.kernel,
        out_type=[jax.ShapeDtypeStruct((N, S), jnp.float32)] * 2,
        mesh=mesh,
        scratch_types=[
            pltpu.VMEM((PH, K), jnp.int32),
            pltpu.VMEM((PH, K), jnp.int32),
            pltpu.VMEM((K, S), jnp.float32),
            pltpu.VMEM((K, S), jnp.float32),
            pltpu.VMEM_SHARED((N, S), jnp.float32),
            pltpu.SemaphoreType.DMA,
            pltpu.SemaphoreType.DMA,
        ],
    )
    def sc(p0, p1, q0, q1, srcr, dstr, out0, out1,
           srcv, dstv, rows0, rows1, acc, sem0, sem1):
        c = lax.axis_index("c")
        s = lax.axis_index("s")
        r0 = pl.multiple_of(s * R_MAIN, 8)

        def rowwise(fn):
            @pl.when(s < NSUB - 1)
            def _():
                fn(r0, R_MAIN)

            @pl.when(s == NSUB - 1)
            def _():
                fn((NSUB - 1) * R_MAIN, R_LAST)

        def init(row0, nrows):
            @pl.when(c == 0)
            def _():
                pltpu.sync_copy(q0.at[pl.ds(row0, nrows)],
                                acc.at[pl.ds(row0, nrows)])

            @pl.when(c == 1)
            def _():
                pltpu.sync_copy(q1.at[pl.ds(row0, nrows)],
                                acc.at[pl.ds(row0, nrows)])

        rowwise(init)
        plsc.subcore_barrier()

        for ps in range(NPASS):
            pltpu.sync_copy(srcr.at[s, ps], srcv)
            pltpu.sync_copy(dstr.at[s, ps], dstv)

            @pl.when(c == 0)
            def _():
                _edge_loop_db(p0, srcv, dstv, acc, rows0, rows1,
                              sem0, sem1, PH)

            @pl.when(c == 1)
            def _():
                _edge_loop_db(p1, srcv, dstv, acc, rows0, rows1,
                              sem0, sem1, PH)

        plsc.subcore_barrier()

        def writeout(row0, nrows):
            @pl.when(c == 0)
            def _():
                pltpu.sync_copy(acc.at[pl.ds(row0, nrows)],
                                out0.at[pl.ds(row0, nrows)])

            @pl.when(c == 1)
            def _():
                pltpu.sync_copy(acc.at[pl.ds(row0, nrows)],
                                out1.at[pl.ds(row0, nrows)])

        rowwise(writeout)

    return sc


@functools.cache
def _make_sc_last():
    D = 128
    mesh = plsc.VectorSubcoreMesh(
        core_axis_name="c", subcore_axis_name="s",
        num_cores=NCORE, num_subcores=NSUB)

    @functools.partial(
        pl.kernel,
        out_type=[jax.ShapeDtypeStruct((N, D), jnp.float32)] * 2,
        mesh=mesh,
        scratch_types=[
            pltpu.VMEM((CH2, K), jnp.int32),
            pltpu.VMEM((CH2, K), jnp.int32),
            pltpu.VMEM((K, D), jnp.float32),
            pltpu.VMEM((K, D), jnp.float32),
            pltpu.VMEM_SHARED((N, D), jnp.float32),
            pltpu.SemaphoreType.DMA,
            pltpu.SemaphoreType.DMA,
        ],
    )
    def sc(p, q, z, srcr, dstr, out0, out1,
           srcv, dstv, rows0, rows1, acc, sem0, sem1):
        c = lax.axis_index("c")
        s = lax.axis_index("s")
        r0 = pl.multiple_of(s * R_MAIN, 8)
        w = c * NSUB + s

        def rowwise(fn):
            @pl.when(s < NSUB - 1)
            def _():
                fn(r0, R_MAIN)

            @pl.when(s == NSUB - 1)
            def _():
                fn((NSUB - 1) * R_MAIN, R_LAST)

        pltpu.sync_copy(srcr.at[w], srcv)
        pltpu.sync_copy(dstr.at[w], dstv)

        def init(row0, nrows):
            @pl.when(c == 0)
            def _():
                pltpu.sync_copy(q.at[pl.ds(row0, nrows)],
                                acc.at[pl.ds(row0, nrows)])

            @pl.when(c == 1)
            def _():
                pltpu.sync_copy(z.at[pl.ds(row0, nrows)],
                                acc.at[pl.ds(row0, nrows)])

        rowwise(init)
        plsc.subcore_barrier()

        _edge_loop_db(p, srcv, dstv, acc, rows0, rows1, sem0, sem1, CH2)
        plsc.subcore_barrier()

        def writeout(row0, nrows):
            @pl.when(c == 0)
            def _():
                pltpu.sync_copy(acc.at[pl.ds(row0, nrows)],
                                out0.at[pl.ds(row0, nrows)])

            @pl.when(c == 1)
            def _():
                pltpu.sync_copy(acc.at[pl.ds(row0, nrows)],
                                out1.at[pl.ds(row0, nrows)])

        rowwise(writeout)

    return sc


def _fin_call(o0, o1, last):
    if last:
        Dout = o0.shape[1]
    else:
        Dout = o0.shape[1] * 2

    def body(a_ref, b_ref, o_ref):
        if last:
            v = a_ref[...] + b_ref[...]
        else:
            v = jnp.concatenate([a_ref[...], b_ref[...]], axis=1)
        nrm = jnp.sqrt(jnp.sum(v * v, axis=1, keepdims=True))
        v = v / jnp.maximum(nrm, 1e-12)
        if last:
            m = jnp.max(v, axis=1, keepdims=True)
            e = v - m
            o_ref[...] = e - jnp.log(jnp.sum(jnp.exp(e), axis=1,
                                             keepdims=True))
        else:
            o_ref[...] = jnp.maximum(v, 0.0)

    Sin = o0.shape[1]
    return pl.pallas_call(
        body,
        grid=(N // BN,),
        in_specs=[pl.BlockSpec((BN, Sin), lambda i: (i, 0))] * 2,
        out_specs=pl.BlockSpec((BN, Dout), lambda i: (i, 0)),
        out_shape=jax.ShapeDtypeStruct((N, Dout), jnp.float32),
    )(o0, o1)


def kernel(x, edge_index, Wl0, bl0, Wr0, Wl1, bl1, Wr1,
           Wl2, bl2, Wr2, Wl3, bl3, Wr3):
    src = edge_index[0].reshape(NSUB, NPASS, PH, K)
    dst = edge_index[1].reshape(NSUB, NPASS, PH, K)
    src2 = edge_index[0].reshape(NCORE * NSUB, CH2, K)
    dst2 = edge_index[1].reshape(NCORE * NSUB, CH2, K)
    zeros = jnp.zeros((N, 128), jnp.float32)
    layers = [(Wl0, bl0, Wr0), (Wl1, bl1, Wr1), (Wl2, bl2, Wr2),
              (Wl3, bl3, Wr3)]
    wcats = [jnp.concatenate([Wl, Wr], axis=1) for Wl, _, Wr in layers]
    b2s = [bl.reshape(1, Wl.shape[1]) for Wl, bl, _ in layers]

    p0, p1, q0, q1 = _mm_call(x, wcats[0], b2s[0], split=True)
    for i in range(3):
        out0, out1 = _make_sc(128)(p0, p1, q0, q1, src, dst)
        if i < 2:
            p0, p1, q0, q1 = _fused_call(out0, out1, wcats[i + 1],
                                         b2s[i + 1], split=True)
        else:
            p, q = _fused_call(out0, out1, wcats[3], b2s[3], split=False)
    out0, out1 = _make_sc_last()(p, q, zeros, src2, dst2)
    return _fin_call(out0, out1, last=True)


# speedup vs baseline: 1.1224x; 1.0410x over previous
import functools

import jax
import jax.numpy as jnp
from jax import lax
from jax.experimental import pallas as pl
from jax.experimental.pallas import tpu as pltpu
from jax.experimental.pallas import tpu_sc as plsc

N = 10000
E = 160000
NCORE = 2
NSUB = 16
K = 125
PH = 40
NPASS = E // NSUB // K // PH
CH2 = E // (NCORE * NSUB) // K
R_MAIN = 624
R_LAST = N - (NSUB - 1) * R_MAIN
BN = 2000


def _mm_call(h, wcat, b2, split):
    Din = h.shape[1]
    Dout = wcat.shape[1] // 2
    S = Dout // 2

    def body(h_ref, w_ref, b_ref, *outs):
        out = jnp.dot(h_ref[...], w_ref[...],
                      preferred_element_type=jnp.float32)
        p = out[:, :Dout]
        q = out[:, Dout:] + b_ref[...]
        if split:
            outs[0][...] = p[:, :S]
            outs[1][...] = p[:, S:]
            outs[2][...] = q[:, :S]
            outs[3][...] = q[:, S:]
        else:
            outs[0][...] = p
            outs[1][...] = q

    n_out, w_out = (4, S) if split else (2, Dout)
    os_ = jax.ShapeDtypeStruct((N, w_out), jnp.float32)
    return pl.pallas_call(
        body,
        grid=(N // BN,),
        in_specs=[
            pl.BlockSpec((BN, Din), lambda i: (i, 0)),
            pl.BlockSpec((Din, 2 * Dout), lambda i: (0, 0)),
            pl.BlockSpec((1, Dout), lambda i: (0, 0)),
        ],
        out_specs=[pl.BlockSpec((BN, w_out), lambda i: (i, 0))] * n_out,
        out_shape=[os_] * n_out,
    )(h, wcat, b2)


def _fused_call(o0, o1, wcat, b2, split):
    Dh = o0.shape[1] * 2
    Dout = wcat.shape[1] // 2
    S = Dout // 2

    def body(a_ref, b_ref, w_ref, bias_ref, *outs):
        v = jnp.concatenate([a_ref[...], b_ref[...]], axis=1)
        nrm = jnp.sqrt(jnp.sum(v * v, axis=1, keepdims=True))
        v = jnp.maximum(v / jnp.maximum(nrm, 1e-12), 0.0)
        out = jnp.dot(v, w_ref[...], preferred_element_type=jnp.float32)
        p = out[:, :Dout]
        q = out[:, Dout:] + bias_ref[...]
        if split:
            outs[0][...] = p[:, :S]
            outs[1][...] = p[:, S:]
            outs[2][...] = q[:, :S]
            outs[3][...] = q[:, S:]
        else:
            outs[0][...] = p
            outs[1][...] = q

    n_out, w_out = (4, S) if split else (2, Dout)
    os_ = jax.ShapeDtypeStruct((N, w_out), jnp.float32)
    return pl.pallas_call(
        body,
        grid=(N // BN,),
        in_specs=[
            pl.BlockSpec((BN, Dh // 2), lambda i: (i, 0)),
            pl.BlockSpec((BN, Dh // 2), lambda i: (i, 0)),
            pl.BlockSpec((Dh, 2 * Dout), lambda i: (0, 0)),
            pl.BlockSpec((1, Dout), lambda i: (0, 0)),
        ],
        out_specs=[pl.BlockSpec((BN, w_out), lambda i: (i, 0))] * n_out,
        out_shape=[os_] * n_out,
    )(o0, o1, wcat, b2)


def _edge_loop_db(tbl, srcv, dstv, acc, rows0, rows1, sem0, sem1, n_chunks):
    pltpu.async_copy(tbl.at[srcv.at[0]], rows0, sem0)

    def body(t, carry):
        a = 2 * t
        pltpu.async_copy(tbl.at[srcv.at[a + 1]], rows1, sem1)
        pltpu.make_async_copy(tbl.at[srcv.at[0]], rows0, sem0).wait()
        pltpu.sync_copy(rows0, acc.at[dstv.at[a]], add=True)

        @pl.when(t < n_chunks // 2 - 1)
        def _():
            pltpu.async_copy(tbl.at[srcv.at[a + 2]], rows0, sem0)

        pltpu.make_async_copy(tbl.at[srcv.at[0]], rows1, sem1).wait()
        pltpu.sync_copy(rows1, acc.at[dstv.at[a + 1]], add=True)
        return carry

    lax.fori_loop(0, n_chunks // 2, body, 0)


@functools.cache
def _make_sc(S):
    mesh = plsc.VectorSubcoreMesh(
        core_axis_name="c", subcore_axis_name="s",
        num_cores=NCORE, num_subcores=NSUB)

    @functools.partial(
        pl.kernel,
        out_type=[jax.ShapeDtypeStruct((N, S), jnp.float32)] * 2,
        mesh=mesh,
        scratch_types=[
            pltpu.VMEM((PH, K), jnp.int32),
            pltpu.VMEM((PH, K), jnp.int32),
            pltpu.VMEM((K, S), jnp.float32),
            pltpu.VMEM((K, S), jnp.float32),
            pltpu.VMEM_SHARED((N, S), jnp.float32),
            pltpu.SemaphoreType.DMA,
            pltpu.SemaphoreType.DMA,
        ],
    )
    def sc(p0, p1, q0, q1, srcr, dstr, out0, out1,
           srcv, dstv, rows0, rows1, acc, sem0, sem1):
        c = lax.axis_index("c")
        s = lax.axis_index("s")
        r0 = pl.multiple_of(s * R_MAIN, 8)

        def rowwise(fn):
            @pl.when(s < NSUB - 1)
            def _():
                fn(r0, R_MAIN)

            @pl.when(s == NSUB - 1)
            def _():
                fn((NSUB - 1) * R_MAIN, R_LAST)

        def init_start(row0, nrows):
            @pl.when(c == 0)
            def _():
                pltpu.async_copy(q0.at[pl.ds(row0, nrows)],
                                 acc.at[pl.ds(row0, nrows)], sem0)

            @pl.when(c == 1)
            def _():
                pltpu.async_copy(q1.at[pl.ds(row0, nrows)],
                                 acc.at[pl.ds(row0, nrows)], sem0)

        def init_wait(row0, nrows):
            @pl.when(c == 0)
            def _():
                pltpu.make_async_copy(q0.at[pl.ds(row0, nrows)],
                                      acc.at[pl.ds(row0, nrows)],
                                      sem0).wait()

            @pl.when(c == 1)
            def _():
                pltpu.make_async_copy(q1.at[pl.ds(row0, nrows)],
                                      acc.at[pl.ds(row0, nrows)],
                                      sem0).wait()

        rowwise(init_start)

        for ps in range(NPASS):
            pltpu.sync_copy(srcr.at[s, ps], srcv)
            pltpu.sync_copy(dstr.at[s, ps], dstv)
            if ps == 0:
                rowwise(init_wait)
                plsc.subcore_barrier()

            @pl.when(c == 0)
            def _():
                _edge_loop_db(p0, srcv, dstv, acc, rows0, rows1,
                              sem0, sem1, PH)

            @pl.when(c == 1)
            def _():
                _edge_loop_db(p1, srcv, dstv, acc, rows0, rows1,
                              sem0, sem1, PH)

        plsc.subcore_barrier()

        def writeout(row0, nrows):
            @pl.when(c == 0)
            def _():
                pltpu.sync_copy(acc.at[pl.ds(row0, nrows)],
                                out0.at[pl.ds(row0, nrows)])

            @pl.when(c == 1)
            def _():
                pltpu.sync_copy(acc.at[pl.ds(row0, nrows)],
                                out1.at[pl.ds(row0, nrows)])

        rowwise(writeout)

    return sc


@functools.cache
def _make_sc_last():
    D = 128
    mesh = plsc.VectorSubcoreMesh(
        core_axis_name="c", subcore_axis_name="s",
        num_cores=NCORE, num_subcores=NSUB)

    @functools.partial(
        pl.kernel,
        out_type=[jax.ShapeDtypeStruct((N, D), jnp.float32)] * 2,
        mesh=mesh,
        scratch_types=[
            pltpu.VMEM((CH2, K), jnp.int32),
            pltpu.VMEM((CH2, K), jnp.int32),
            pltpu.VMEM((K, D), jnp.float32),
            pltpu.VMEM((K, D), jnp.float32),
            pltpu.VMEM_SHARED((N, D), jnp.float32),
            pltpu.SemaphoreType.DMA,
            pltpu.SemaphoreType.DMA,
        ],
    )
    def sc(p, q, z, srcr, dstr, out0, out1,
           srcv, dstv, rows0, rows1, acc, sem0, sem1):
        c = lax.axis_index("c")
        s = lax.axis_index("s")
        r0 = pl.multiple_of(s * R_MAIN, 8)
        w = c * NSUB + s

        def rowwise(fn):
            @pl.when(s < NSUB - 1)
            def _():
                fn(r0, R_MAIN)

            @pl.when(s == NSUB - 1)
            def _():
                fn((NSUB - 1) * R_MAIN, R_LAST)

        def init_start(row0, nrows):
            @pl.when(c == 0)
            def _():
                pltpu.async_copy(q.at[pl.ds(row0, nrows)],
                                 acc.at[pl.ds(row0, nrows)], sem0)

            @pl.when(c == 1)
            def _():
                pltpu.async_copy(z.at[pl.ds(row0, nrows)],
                                 acc.at[pl.ds(row0, nrows)], sem0)

        def init_wait(row0, nrows):
            @pl.when(c == 0)
            def _():
                pltpu.make_async_copy(q.at[pl.ds(row0, nrows)],
                                      acc.at[pl.ds(row0, nrows)],
                                      sem0).wait()

            @pl.when(c == 1)
            def _():
                pltpu.make_async_copy(z.at[pl.ds(row0, nrows)],
                                      acc.at[pl.ds(row0, nrows)],
                                      sem0).wait()

        rowwise(init_start)
        pltpu.sync_copy(srcr.at[w], srcv)
        pltpu.sync_copy(dstr.at[w], dstv)
        rowwise(init_wait)
        plsc.subcore_barrier()

        _edge_loop_db(p, srcv, dstv, acc, rows0, rows1, sem0, sem1, CH2)
        plsc.subcore_barrier()

        def writeout(row0, nrows):
            @pl.when(c == 0)
            def _():
                pltpu.sync_copy(acc.at[pl.ds(row0, nrows)],
                                out0.at[pl.ds(row0, nrows)])

            @pl.when(c == 1)
            def _():
                pltpu.sync_copy(acc.at[pl.ds(row0, nrows)],
                                out1.at[pl.ds(row0, nrows)])

        rowwise(writeout)

    return sc


def _fin_call(o0, o1, last):
    if last:
        Dout = o0.shape[1]
    else:
        Dout = o0.shape[1] * 2

    def body(a_ref, b_ref, o_ref):
        if last:
            v = a_ref[...] + b_ref[...]
        else:
            v = jnp.concatenate([a_ref[...], b_ref[...]], axis=1)
        nrm = jnp.sqrt(jnp.sum(v * v, axis=1, keepdims=True))
        v = v / jnp.maximum(nrm, 1e-12)
        if last:
            m = jnp.max(v, axis=1, keepdims=True)
            e = v - m
            o_ref[...] = e - jnp.log(jnp.sum(jnp.exp(e), axis=1,
                                             keepdims=True))
        else:
            o_ref[...] = jnp.maximum(v, 0.0)

    Sin = o0.shape[1]
    return pl.pallas_call(
        body,
        grid=(N // BN,),
        in_specs=[pl.BlockSpec((BN, Sin), lambda i: (i, 0))] * 2,
        out_specs=pl.BlockSpec((BN, Dout), lambda i: (i, 0)),
        out_shape=jax.ShapeDtypeStruct((N, Dout), jnp.float32),
    )(o0, o1)


def kernel(x, edge_index, Wl0, bl0, Wr0, Wl1, bl1, Wr1,
           Wl2, bl2, Wr2, Wl3, bl3, Wr3):
    src = edge_index[0].reshape(NSUB, NPASS, PH, K)
    dst = edge_index[1].reshape(NSUB, NPASS, PH, K)
    src2 = edge_index[0].reshape(NCORE * NSUB, CH2, K)
    dst2 = edge_index[1].reshape(NCORE * NSUB, CH2, K)
    zeros = jnp.zeros((N, 128), jnp.float32)
    layers = [(Wl0, bl0, Wr0), (Wl1, bl1, Wr1), (Wl2, bl2, Wr2),
              (Wl3, bl3, Wr3)]
    wcats = [jnp.concatenate([Wl, Wr], axis=1) for Wl, _, Wr in layers]
    b2s = [bl.reshape(1, Wl.shape[1]) for Wl, bl, _ in layers]

    p0, p1, q0, q1 = _mm_call(x, wcats[0], b2s[0], split=True)
    for i in range(3):
        out0, out1 = _make_sc(128)(p0, p1, q0, q1, src, dst)
        if i < 2:
            p0, p1, q0, q1 = _fused_call(out0, out1, wcats[i + 1],
                                         b2s[i + 1], split=True)
        else:
            p, q = _fused_call(out0, out1, wcats[3], b2s[3], split=False)
    out0, out1 = _make_sc_last()(p, q, zeros, src2, dst2)
    return _fin_call(out0, out1, last=True)
